# baseline (speedup 1.0000x reference)
import functools
import math

import jax
import jax.numpy as jnp
from jax import lax
from jax.experimental import pallas as pl
from jax.experimental.pallas import tpu as pltpu
from jax.experimental.pallas import tpu_sc as plsc

NUM_BUCKETS = 32
NUM_HEADS = 16
MAX_DISTANCE = 128
Q = 2048
K = 2048
GROUP = 128
DW = 3968
VW = 4224
NGROUPS = Q // GROUP
NTEC = 32
DMAS_PER_TEC = NUM_HEADS * NGROUPS // NTEC


def _table_body(wt_ref, out_ref, v_ref):
    t = lax.broadcasted_iota(jnp.int32, (NUM_BUCKETS, VW), 1)
    n = jnp.maximum((Q - 1) - t, 0)
    max_exact = NUM_BUCKETS // 2
    nf = n.astype(jnp.float32)
    val_if_large = max_exact + (
        jnp.log(nf / max_exact + 1e-09)
        / math.log(MAX_DISTANCE / max_exact)
        * (NUM_BUCKETS - max_exact)
    ).astype(jnp.int32)
    val_if_large = jnp.minimum(val_if_large, NUM_BUCKETS - 1)
    bkt = jnp.where(n < max_exact, n, val_if_large)
    b_iota = lax.broadcasted_iota(jnp.int32, (NUM_BUCKETS, VW), 0)
    onehot = (bkt == b_iota).astype(jnp.float32)
    v_ref[...] = lax.dot_general(
        wt_ref[...],
        onehot,
        (((1,), (0,)), ((), ())),
        precision=lax.Precision.HIGHEST,
        preferred_element_type=jnp.float32,
    )
    for i in range(GROUP):
        s = GROUP - 1 - i
        out_ref[:, i, :] = v_ref[:, s : s + DW]


_build_table = pl.pallas_call(
    _table_body,
    out_shape=jax.ShapeDtypeStruct((NUM_HEADS, GROUP, DW), jnp.float32),
    scratch_shapes=[pltpu.VMEM((NUM_HEADS, VW), jnp.float32)],
)


def _expand_body(dsh_hbm, out_hbm, sem):
    c = lax.axis_index("c")
    s = lax.axis_index("s")
    wid = s * 2 + c

    def fire(gi, carry):
        g = wid * DMAS_PER_TEC + gi
        h = g // NGROUPS
        q0 = (g - h * NGROUPS) * GROUP
        b = (Q - GROUP) - q0
        pltpu.make_async_copy(
            dsh_hbm.at[h, :, pl.ds(b, K)],
            out_hbm.at[0, h, pl.ds(q0, GROUP), :],
            sem,
        ).start()
        return carry

    lax.fori_loop(0, DMAS_PER_TEC, fire, 0)

    def drain(gi, carry):
        pltpu.make_async_copy(
            dsh_hbm.at[0, :, pl.ds(0, K)],
            out_hbm.at[0, 0, pl.ds(0, GROUP), :],
            sem,
        ).wait()
        return carry

    lax.fori_loop(0, DMAS_PER_TEC, drain, 0)


@functools.cache
def _expand():
    return pl.kernel(
        _expand_body,
        out_type=jax.ShapeDtypeStruct((1, NUM_HEADS, Q, K), jnp.float32),
        mesh=plsc.VectorSubcoreMesh(core_axis_name="c", subcore_axis_name="s"),
        scratch_types=[pltpu.SemaphoreType.DMA],
    )


def kernel(qlen, klen, W):
    wt = W.T
    dsh = _build_table(wt)
    return _expand()(dsh)


# speedup vs baseline: 41.6985x; 41.6985x over previous
import functools
import math

import jax
import jax.numpy as jnp
from jax import lax
from jax.experimental import pallas as pl
from jax.experimental.pallas import tpu as pltpu
from jax.experimental.pallas import tpu_sc as plsc

NUM_BUCKETS = 32
NUM_HEADS = 16
MAX_DISTANCE = 128
Q = 2048
K = 2048
GROUP = 128
DW = 3968
VW = 4224
NGROUPS = Q // GROUP
NTEC = 32
DMAS_PER_TEC = NUM_HEADS * NGROUPS // NTEC


def _table_body(wt_ref, out_ref, v_ref):
    t = lax.broadcasted_iota(jnp.int32, (NUM_BUCKETS, VW), 1)
    n = jnp.maximum((Q - 1) - t, 0)
    max_exact = NUM_BUCKETS // 2
    nf = n.astype(jnp.float32)
    val_if_large = max_exact + (
        jnp.log(nf / max_exact + 1e-09)
        / math.log(MAX_DISTANCE / max_exact)
        * (NUM_BUCKETS - max_exact)
    ).astype(jnp.int32)
    val_if_large = jnp.minimum(val_if_large, NUM_BUCKETS - 1)
    bkt = jnp.where(n < max_exact, n, val_if_large)
    b_iota = lax.broadcasted_iota(jnp.int32, (NUM_BUCKETS, VW), 0)
    onehot = (bkt == b_iota).astype(jnp.float32)
    v_ref[...] = lax.dot_general(
        wt_ref[...],
        onehot,
        (((1,), (0,)), ((), ())),
        precision=lax.Precision.HIGHEST,
        preferred_element_type=jnp.float32,
    )
    for i in range(GROUP):
        s = GROUP - 1 - i
        out_ref[:, i, :] = v_ref[:, s : s + DW]


_build_table = pl.pallas_call(
    _table_body,
    out_shape=jax.ShapeDtypeStruct((NUM_HEADS, GROUP, DW), jnp.float32),
    scratch_shapes=[pltpu.VMEM((NUM_HEADS, VW), jnp.float32)],
)


HEADS_PER_WAVE = 4
WAVES = NUM_HEADS // 2 // HEADS_PER_WAVE
WROWS = HEADS_PER_WAVE * GROUP
STAGE_ROWS = WROWS // 16
GROUPS_PER_TEC_WAVE = HEADS_PER_WAVE * NGROUPS // 16


def _expand_body(dsh_hbm, out_hbm, shared, sem):
    c = lax.axis_index("c")
    s = lax.axis_index("s")

    for wave in range(WAVES):
        base_h = c * (NUM_HEADS // 2) + wave * HEADS_PER_WAVE
        r0 = base_h * GROUP + s * STAGE_ROWS
        pltpu.sync_copy(
            dsh_hbm.at[pl.ds(r0, STAGE_ROWS), :],
            shared.at[pl.ds(s * STAGE_ROWS, STAGE_ROWS), :],
        )
        plsc.subcore_barrier()

        for j in range(GROUPS_PER_TEC_WAVE):
            g = s * GROUPS_PER_TEC_WAVE + j
            hw = g // NGROUPS
            q0 = (g - hw * NGROUPS) * GROUP
            b = (Q - GROUP) - q0
            pltpu.make_async_copy(
                shared.at[pl.ds(hw * GROUP, GROUP), pl.ds(b, K)],
                out_hbm.at[0, base_h + hw, pl.ds(q0, GROUP), :],
                sem,
            ).start()

        for j in range(GROUPS_PER_TEC_WAVE):
            pltpu.make_async_copy(
                shared.at[pl.ds(0, GROUP), pl.ds(0, K)],
                out_hbm.at[0, 0, pl.ds(0, GROUP), :],
                sem,
            ).wait()
        plsc.subcore_barrier()


@functools.cache
def _expand():
    return pl.kernel(
        _expand_body,
        out_type=jax.ShapeDtypeStruct((1, NUM_HEADS, Q, K), jnp.float32),
        mesh=plsc.VectorSubcoreMesh(core_axis_name="c", subcore_axis_name="s"),
        scratch_types=[
            pltpu.VMEM_SHARED((WROWS, DW), jnp.float32),
            pltpu.SemaphoreType.DMA,
        ],
    )


def kernel(qlen, klen, W):
    wt = W.T
    dsh = _build_table(wt)
    dsh2 = dsh.reshape(NUM_HEADS * GROUP, DW)
    return _expand()(dsh2)
